# baseline (speedup 1.0000x reference)
import functools

import jax
import jax.numpy as jnp
from jax import lax
from jax.experimental import pallas as pl
from jax.experimental.pallas import tpu as pltpu
from jax.experimental.pallas import tpu_sc as plsc

B = 4096
L = 50
E = 128
HIDDEN = 4096
CLASSES = 1024

NUM_CORES = 2
NUM_SUBCORES = 16
NW = NUM_CORES * NUM_SUBCORES
CHUNK = 4
NBUF = 4
LANES = 16

SPLITS = (1024, 1536, 1536)
BB = 512


def _make_pool_body(rpw, nch, chunk_base):
    def _pool_body(idx_hbm, table_hbm, out_hbm, idx_v,
                   rows0_v, rows1_v, rows2_v, rows3_v, out_v,
                   sem0, sem1, sem2, sem3):
        wid = lax.axis_index("s") * NUM_CORES + lax.axis_index("c")
        base = chunk_base + wid * (rpw * L)
        pltpu.sync_copy(idx_hbm.at[pl.ds(base, rpw * L)], idx_v)

        bufs = (rows0_v, rows1_v, rows2_v, rows3_v)
        sems = (sem0, sem1, sem2, sem3)

        def start(g, b):
            pltpu.make_async_copy(
                table_hbm.at[idx_v.at[pl.ds(g * (CHUNK * L), CHUNK * L)]],
                bufs[b], sems[b],
            ).start()

        def wait(b):
            pltpu.make_async_copy(
                table_hbm.at[idx_v.at[pl.ds(0, CHUNK * L)]],
                bufs[b], sems[b],
            ).wait()

        def reduce_chunk(g, b):
            rows_v = bufs[b]
            for j in range(CHUNK):
                def body(r, accs):
                    return tuple(
                        a + rows_v[j * L + r, pl.ds(c * LANES, LANES)]
                        for c, a in enumerate(accs)
                    )
                accs = tuple(
                    jnp.zeros((LANES,), jnp.float32) for _ in range(E // LANES))
                accs = lax.fori_loop(0, L, body, accs, unroll=5)
                for c in range(E // LANES):
                    out_v[g * CHUNK + j, pl.ds(c * LANES, LANES)] = (
                        accs[c] * (1.0 / L))

        for b in range(NBUF):
            start(b, b)

        @pl.loop(0, nch, step=NBUF)
        def _(g):
            for b in range(NBUF):
                gg = g + b
                wait(b)
                reduce_chunk(gg, b)

                @pl.when(gg + NBUF < nch)
                def _():
                    start(gg + NBUF, b)

        pltpu.sync_copy(out_v, out_hbm.at[pl.ds(wid * rpw, rpw)])

    return _pool_body


def _pool(idx, table, chunk_base, nb):
    rpw = nb // NW
    nch = rpw // CHUNK
    k = pl.kernel(
        _make_pool_body(rpw, nch, chunk_base),
        out_type=jax.ShapeDtypeStruct((nb, E), jnp.float32),
        mesh=plsc.VectorSubcoreMesh(core_axis_name="c", subcore_axis_name="s"),
        scratch_types=[
            pltpu.VMEM((rpw * L,), jnp.int32),
            pltpu.VMEM((CHUNK * L, E), jnp.float32),
            pltpu.VMEM((CHUNK * L, E), jnp.float32),
            pltpu.VMEM((CHUNK * L, E), jnp.float32),
            pltpu.VMEM((CHUNK * L, E), jnp.float32),
            pltpu.VMEM((rpw, E), jnp.float32),
            pltpu.SemaphoreType.DMA,
            pltpu.SemaphoreType.DMA,
            pltpu.SemaphoreType.DMA,
            pltpu.SemaphoreType.DMA,
        ],
    )
    return k(idx, table)


def _mlp_body(acc_ref, p_ref, w1_ref, b1_ref, w2_ref, b2_ref, o_ref):
    del acc_ref
    h = jnp.dot(p_ref[...], w1_ref[...], preferred_element_type=jnp.float32)
    h = jnp.maximum(h + b1_ref[...], 0.0)
    logits = jnp.dot(h, w2_ref[...], preferred_element_type=jnp.float32)
    logits = logits + b2_ref[...]
    m = jnp.max(logits, axis=1, keepdims=True)
    s = logits - m
    lse = jnp.log(jnp.sum(jnp.exp(s), axis=1, keepdims=True))
    o_ref[...] = s - lse


def _mlp_into(acc, pooled, W1, b1, W2, b2, block_off):
    nb = pooled.shape[0]
    weight_specs = [
        pl.BlockSpec((E, HIDDEN), lambda i: (0, 0)),
        pl.BlockSpec((1, HIDDEN), lambda i: (0, 0)),
        pl.BlockSpec((HIDDEN, CLASSES), lambda i: (0, 0)),
        pl.BlockSpec((1, CLASSES), lambda i: (0, 0)),
    ]
    pooled_spec = pl.BlockSpec((BB, E), lambda i: (i, 0))
    out_spec = pl.BlockSpec((BB, CLASSES), lambda i: (block_off + i, 0))
    out_shape = jax.ShapeDtypeStruct((B, CLASSES), jnp.float32)
    if acc is None:
        body = functools.partial(_mlp_body, None)
        return pl.pallas_call(
            body,
            grid=(nb // BB,),
            in_specs=[pooled_spec] + weight_specs,
            out_specs=out_spec,
            out_shape=out_shape,
        )(pooled, W1, b1, W2, b2)
    return pl.pallas_call(
        _mlp_body,
        grid=(nb // BB,),
        in_specs=[pl.BlockSpec(memory_space=pl.ANY), pooled_spec] + weight_specs,
        out_specs=out_spec,
        out_shape=out_shape,
        input_output_aliases={0: 0},
    )(acc, pooled, W1, b1, W2, b2)


@jax.jit
def _run(idx, emb_table, W1, b1, W2, b2):
    b1r = b1.reshape(1, HIDDEN)
    b2r = b2.reshape(1, CLASSES)
    out = None
    row0 = 0
    for bc in SPLITS:
        pooled = _pool(idx, emb_table, row0 * L, bc)
        out = _mlp_into(out, pooled, W1, b1r, W2, b2r, row0 // BB)
        row0 += bc
    return out


def kernel(x, emb_table, W1, b1, W2, b2):
    idx = x.reshape(-1)
    if idx.dtype != jnp.int32:
        idx = idx.astype(jnp.int32)
    return _run(idx, emb_table, W1, b1, W2, b2)


# speedup vs baseline: 1.0324x; 1.0324x over previous
import functools

import jax
import jax.numpy as jnp
from jax import lax
from jax.experimental import pallas as pl
from jax.experimental.pallas import tpu as pltpu
from jax.experimental.pallas import tpu_sc as plsc

B = 4096
L = 50
E = 128
HIDDEN = 4096
CLASSES = 1024

NUM_CORES = 2
NUM_SUBCORES = 16
NW = NUM_CORES * NUM_SUBCORES
CHUNK = 4
NBUF = 4
LANES = 16

SPLITS = (2048, 2048)
BB = 512


def _make_pool_body(rpw, nch, chunk_base):
    def _pool_body(idx_hbm, table_hbm, out_hbm, idx_v,
                   rows0_v, rows1_v, rows2_v, rows3_v, out_v,
                   sem0, sem1, sem2, sem3):
        wid = lax.axis_index("s") * NUM_CORES + lax.axis_index("c")
        base = chunk_base + wid * (rpw * L)
        pltpu.sync_copy(idx_hbm.at[pl.ds(base, rpw * L)], idx_v)

        bufs = (rows0_v, rows1_v, rows2_v, rows3_v)
        sems = (sem0, sem1, sem2, sem3)

        def start(g, b):
            pltpu.make_async_copy(
                table_hbm.at[idx_v.at[pl.ds(g * (CHUNK * L), CHUNK * L)]],
                bufs[b], sems[b],
            ).start()

        def wait(b):
            pltpu.make_async_copy(
                table_hbm.at[idx_v.at[pl.ds(0, CHUNK * L)]],
                bufs[b], sems[b],
            ).wait()

        def reduce_chunk(g, b):
            rows_v = bufs[b]
            for j in range(CHUNK):
                def body(r, accs):
                    return tuple(
                        a + rows_v[j * L + r, pl.ds(c * LANES, LANES)]
                        for c, a in enumerate(accs)
                    )
                accs = tuple(
                    jnp.zeros((LANES,), jnp.float32) for _ in range(E // LANES))
                accs = lax.fori_loop(0, L, body, accs, unroll=5)
                for c in range(E // LANES):
                    out_v[g * CHUNK + j, pl.ds(c * LANES, LANES)] = (
                        accs[c] * (1.0 / L))

        for b in range(NBUF):
            start(b, b)

        @pl.loop(0, nch, step=NBUF)
        def _(g):
            for b in range(NBUF):
                gg = g + b
                wait(b)
                reduce_chunk(gg, b)

                @pl.when(gg + NBUF < nch)
                def _():
                    start(gg + NBUF, b)

        pltpu.sync_copy(out_v, out_hbm.at[pl.ds(wid * rpw, rpw)])

    return _pool_body


def _pool(idx, table, chunk_base, nb):
    rpw = nb // NW
    nch = rpw // CHUNK
    k = pl.kernel(
        _make_pool_body(rpw, nch, chunk_base),
        out_type=jax.ShapeDtypeStruct((nb, E), jnp.float32),
        mesh=plsc.VectorSubcoreMesh(core_axis_name="c", subcore_axis_name="s"),
        scratch_types=[
            pltpu.VMEM((rpw * L,), jnp.int32),
            pltpu.VMEM((CHUNK * L, E), jnp.float32),
            pltpu.VMEM((CHUNK * L, E), jnp.float32),
            pltpu.VMEM((CHUNK * L, E), jnp.float32),
            pltpu.VMEM((CHUNK * L, E), jnp.float32),
            pltpu.VMEM((rpw, E), jnp.float32),
            pltpu.SemaphoreType.DMA,
            pltpu.SemaphoreType.DMA,
            pltpu.SemaphoreType.DMA,
            pltpu.SemaphoreType.DMA,
        ],
    )
    return k(idx, table)


def _mlp_body(acc_ref, p_ref, w1_ref, b1_ref, w2_ref, b2_ref, o_ref):
    del acc_ref
    h = jnp.dot(p_ref[...].astype(jnp.bfloat16), w1_ref[...],
                preferred_element_type=jnp.float32)
    h = jnp.maximum(h + b1_ref[...], 0.0).astype(jnp.bfloat16)
    logits = jnp.dot(h, w2_ref[...], preferred_element_type=jnp.float32)
    logits = logits + b2_ref[...]
    m = jnp.max(logits, axis=1, keepdims=True)
    s = logits - m
    lse = jnp.log(jnp.sum(jnp.exp(s), axis=1, keepdims=True))
    o_ref[...] = s - lse


def _mlp_into(acc, pooled, W1, b1, W2, b2, block_off):
    nb = pooled.shape[0]
    weight_specs = [
        pl.BlockSpec((E, HIDDEN), lambda i: (0, 0)),
        pl.BlockSpec((1, HIDDEN), lambda i: (0, 0)),
        pl.BlockSpec((HIDDEN, CLASSES), lambda i: (0, 0)),
        pl.BlockSpec((1, CLASSES), lambda i: (0, 0)),
    ]
    pooled_spec = pl.BlockSpec((BB, E), lambda i: (i, 0))
    out_spec = pl.BlockSpec((BB, CLASSES), lambda i: (block_off + i, 0))
    out_shape = jax.ShapeDtypeStruct((B, CLASSES), jnp.float32)
    if acc is None:
        body = functools.partial(_mlp_body, None)
        return pl.pallas_call(
            body,
            grid=(nb // BB,),
            in_specs=[pooled_spec] + weight_specs,
            out_specs=out_spec,
            out_shape=out_shape,
        )(pooled, W1, b1, W2, b2)
    return pl.pallas_call(
        _mlp_body,
        grid=(nb // BB,),
        in_specs=[pl.BlockSpec(memory_space=pl.ANY), pooled_spec] + weight_specs,
        out_specs=out_spec,
        out_shape=out_shape,
        input_output_aliases={0: 0},
    )(acc, pooled, W1, b1, W2, b2)


@jax.jit
def _run(idx, emb_table, W1, b1, W2, b2):
    b1r = b1.reshape(1, HIDDEN)
    b2r = b2.reshape(1, CLASSES)
    W1b = W1.astype(jnp.bfloat16)
    W2b = W2.astype(jnp.bfloat16)
    out = None
    row0 = 0
    for bc in SPLITS:
        pooled = _pool(idx, emb_table, row0 * L, bc)
        out = _mlp_into(out, pooled, W1b, b1r, W2b, b2r, row0 // BB)
        row0 += bc
    return out


def kernel(x, emb_table, W1, b1, W2, b2):
    idx = x.reshape(-1)
    if idx.dtype != jnp.int32:
        idx = idx.astype(jnp.int32)
    return _run(idx, emb_table, W1, b1, W2, b2)


# speedup vs baseline: 1.0608x; 1.0275x over previous
import functools

import jax
import jax.numpy as jnp
from jax import lax
from jax.experimental import pallas as pl
from jax.experimental.pallas import tpu as pltpu
from jax.experimental.pallas import tpu_sc as plsc

B = 4096
L = 50
E = 128
HIDDEN = 4096
CLASSES = 1024

NUM_CORES = 2
NUM_SUBCORES = 16
NW = NUM_CORES * NUM_SUBCORES
CHUNK = 4
NBUF = 4
LANES = 16

SPLITS = (2048, 2048)
BB = 1024


def _make_pool_body(rpw, nch, chunk_base):
    def _pool_body(idx_hbm, table_hbm, out_hbm, idx_v,
                   rows0_v, rows1_v, rows2_v, rows3_v, out_v,
                   sem0, sem1, sem2, sem3):
        wid = lax.axis_index("s") * NUM_CORES + lax.axis_index("c")
        base = chunk_base + wid * (rpw * L)
        pltpu.sync_copy(idx_hbm.at[pl.ds(base, rpw * L)], idx_v)

        bufs = (rows0_v, rows1_v, rows2_v, rows3_v)
        sems = (sem0, sem1, sem2, sem3)

        def start(g, b):
            pltpu.make_async_copy(
                table_hbm.at[idx_v.at[pl.ds(g * (CHUNK * L), CHUNK * L)]],
                bufs[b], sems[b],
            ).start()

        def wait(b):
            pltpu.make_async_copy(
                table_hbm.at[idx_v.at[pl.ds(0, CHUNK * L)]],
                bufs[b], sems[b],
            ).wait()

        def reduce_chunk(g, b):
            rows_v = bufs[b]
            for j in range(CHUNK):
                def body(r, accs):
                    return tuple(
                        a + rows_v[j * L + r, pl.ds(c * LANES, LANES)]
                        for c, a in enumerate(accs)
                    )
                accs = tuple(
                    jnp.zeros((LANES,), jnp.float32) for _ in range(E // LANES))
                accs = lax.fori_loop(0, L, body, accs, unroll=5)
                for c in range(E // LANES):
                    out_v[g * CHUNK + j, pl.ds(c * LANES, LANES)] = (
                        accs[c] * (1.0 / L))

        for b in range(NBUF):
            start(b, b)

        @pl.loop(0, nch, step=NBUF)
        def _(g):
            for b in range(NBUF):
                gg = g + b
                wait(b)
                reduce_chunk(gg, b)

                @pl.when(gg + NBUF < nch)
                def _():
                    start(gg + NBUF, b)

        pltpu.sync_copy(out_v, out_hbm.at[pl.ds(wid * rpw, rpw)])

    return _pool_body


def _pool(idx, table, chunk_base, nb):
    rpw = nb // NW
    nch = rpw // CHUNK
    k = pl.kernel(
        _make_pool_body(rpw, nch, chunk_base),
        out_type=jax.ShapeDtypeStruct((nb, E), jnp.float32),
        mesh=plsc.VectorSubcoreMesh(core_axis_name="c", subcore_axis_name="s"),
        scratch_types=[
            pltpu.VMEM((rpw * L,), jnp.int32),
            pltpu.VMEM((CHUNK * L, E), jnp.float32),
            pltpu.VMEM((CHUNK * L, E), jnp.float32),
            pltpu.VMEM((CHUNK * L, E), jnp.float32),
            pltpu.VMEM((CHUNK * L, E), jnp.float32),
            pltpu.VMEM((rpw, E), jnp.float32),
            pltpu.SemaphoreType.DMA,
            pltpu.SemaphoreType.DMA,
            pltpu.SemaphoreType.DMA,
            pltpu.SemaphoreType.DMA,
        ],
    )
    return k(idx, table)


def _mlp_body(acc_ref, p_ref, w1_ref, b1_ref, w2_ref, b2_ref, o_ref):
    del acc_ref
    h = jnp.dot(p_ref[...], w1_ref[...], preferred_element_type=jnp.float32)
    h = jnp.maximum(h + b1_ref[...], 0.0)
    logits = jnp.dot(h, w2_ref[...], preferred_element_type=jnp.float32)
    logits = logits + b2_ref[...]
    m = jnp.max(logits, axis=1, keepdims=True)
    s = logits - m
    lse = jnp.log(jnp.sum(jnp.exp(s), axis=1, keepdims=True))
    o_ref[...] = s - lse


def _mlp_into(acc, pooled, W1, b1, W2, b2, block_off):
    nb = pooled.shape[0]
    weight_specs = [
        pl.BlockSpec((E, HIDDEN), lambda i: (0, 0)),
        pl.BlockSpec((1, HIDDEN), lambda i: (0, 0)),
        pl.BlockSpec((HIDDEN, CLASSES), lambda i: (0, 0)),
        pl.BlockSpec((1, CLASSES), lambda i: (0, 0)),
    ]
    pooled_spec = pl.BlockSpec((BB, E), lambda i: (i, 0))
    out_spec = pl.BlockSpec((BB, CLASSES), lambda i: (block_off + i, 0))
    out_shape = jax.ShapeDtypeStruct((B, CLASSES), jnp.float32)
    if acc is None:
        body = functools.partial(_mlp_body, None)
        return pl.pallas_call(
            body,
            grid=(nb // BB,),
            in_specs=[pooled_spec] + weight_specs,
            out_specs=out_spec,
            out_shape=out_shape,
        )(pooled, W1, b1, W2, b2)
    return pl.pallas_call(
        _mlp_body,
        grid=(nb // BB,),
        in_specs=[pl.BlockSpec(memory_space=pl.ANY), pooled_spec] + weight_specs,
        out_specs=out_spec,
        out_shape=out_shape,
        input_output_aliases={0: 0},
    )(acc, pooled, W1, b1, W2, b2)


@jax.jit
def _run(idx, emb_table, W1, b1, W2, b2):
    b1r = b1.reshape(1, HIDDEN)
    b2r = b2.reshape(1, CLASSES)
    out = None
    row0 = 0
    for bc in SPLITS:
        pooled = _pool(idx, emb_table, row0 * L, bc)
        out = _mlp_into(out, pooled, W1, b1r, W2, b2r, row0 // BB)
        row0 += bc
    return out


def kernel(x, emb_table, W1, b1, W2, b2):
    idx = x.reshape(-1)
    if idx.dtype != jnp.int32:
        idx = idx.astype(jnp.int32)
    return _run(idx, emb_table, W1, b1, W2, b2)


# speedup vs baseline: 1.0647x; 1.0036x over previous
import functools

import jax
import jax.numpy as jnp
from jax import lax
from jax.experimental import pallas as pl
from jax.experimental.pallas import tpu as pltpu
from jax.experimental.pallas import tpu_sc as plsc

B = 4096
L = 50
E = 128
HIDDEN = 4096
CLASSES = 1024

NUM_CORES = 2
NUM_SUBCORES = 16
NW = NUM_CORES * NUM_SUBCORES
CHUNK = 4
NBUF = 4
LANES = 16

SPLITS = (2048, 2048)
BB = 512


def _make_pool_body(rpw, nch, chunk_base):
    def _pool_body(idx_hbm, table_hbm, out_hbm, idx_v,
                   rows0_v, rows1_v, rows2_v, rows3_v, out_v,
                   sem0, sem1, sem2, sem3):
        wid = lax.axis_index("s") * NUM_CORES + lax.axis_index("c")
        base = chunk_base + wid * (rpw * L)
        pltpu.sync_copy(idx_hbm.at[pl.ds(base, rpw * L)], idx_v)

        bufs = (rows0_v, rows1_v, rows2_v, rows3_v)
        sems = (sem0, sem1, sem2, sem3)

        def start(g, b):
            pltpu.make_async_copy(
                table_hbm.at[idx_v.at[pl.ds(g * (CHUNK * L), CHUNK * L)]],
                bufs[b], sems[b],
            ).start()

        def wait(b):
            pltpu.make_async_copy(
                table_hbm.at[idx_v.at[pl.ds(0, CHUNK * L)]],
                bufs[b], sems[b],
            ).wait()

        def reduce_chunk(g, b):
            rows_v = bufs[b]
            for j in range(CHUNK):
                def body(r, accs):
                    return tuple(
                        a + rows_v[j * L + r, pl.ds(c * LANES, LANES)]
                        for c, a in enumerate(accs)
                    )
                accs = tuple(
                    jnp.zeros((LANES,), jnp.float32) for _ in range(E // LANES))
                accs = lax.fori_loop(0, L, body, accs, unroll=5)
                for c in range(E // LANES):
                    out_v[g * CHUNK + j, pl.ds(c * LANES, LANES)] = (
                        accs[c] * (1.0 / L))

        for b in range(NBUF):
            start(b, b)

        @pl.loop(0, nch, step=NBUF)
        def _(g):
            for b in range(NBUF):
                gg = g + b
                wait(b)
                reduce_chunk(gg, b)

                @pl.when(gg + NBUF < nch)
                def _():
                    start(gg + NBUF, b)

        pltpu.sync_copy(out_v, out_hbm.at[pl.ds(wid * rpw, rpw)])

    return _pool_body


def _pool(idx, table, chunk_base, nb):
    rpw = nb // NW
    nch = rpw // CHUNK
    k = pl.kernel(
        _make_pool_body(rpw, nch, chunk_base),
        out_type=jax.ShapeDtypeStruct((nb, E), jnp.float32),
        mesh=plsc.VectorSubcoreMesh(core_axis_name="c", subcore_axis_name="s"),
        scratch_types=[
            pltpu.VMEM((rpw * L,), jnp.int32),
            pltpu.VMEM((CHUNK * L, E), jnp.float32),
            pltpu.VMEM((CHUNK * L, E), jnp.float32),
            pltpu.VMEM((CHUNK * L, E), jnp.float32),
            pltpu.VMEM((CHUNK * L, E), jnp.float32),
            pltpu.VMEM((rpw, E), jnp.float32),
            pltpu.SemaphoreType.DMA,
            pltpu.SemaphoreType.DMA,
            pltpu.SemaphoreType.DMA,
            pltpu.SemaphoreType.DMA,
        ],
    )
    return k(idx, table)


def _mlp_body(acc_ref, p_ref, w1_ref, b1_ref, w2_ref, b2_ref, o_ref):
    del acc_ref
    h = jnp.dot(p_ref[...], w1_ref[...], preferred_element_type=jnp.float32)
    h = jnp.maximum(h + b1_ref[...], 0.0)
    logits = jnp.dot(h, w2_ref[...], preferred_element_type=jnp.float32)
    logits = logits + b2_ref[...]
    m = jnp.max(logits, axis=1, keepdims=True)
    s = logits - m
    lse = jnp.log(jnp.sum(jnp.exp(s), axis=1, keepdims=True))
    o_ref[...] = s - lse


def _mlp_into(acc, pooled, W1, b1, W2, b2, block_off):
    nb = pooled.shape[0]
    weight_specs = [
        pl.BlockSpec((E, HIDDEN), lambda i: (0, 0)),
        pl.BlockSpec((1, HIDDEN), lambda i: (0, 0)),
        pl.BlockSpec((HIDDEN, CLASSES), lambda i: (0, 0)),
        pl.BlockSpec((1, CLASSES), lambda i: (0, 0)),
    ]
    pooled_spec = pl.BlockSpec((BB, E), lambda i: (i, 0))
    out_spec = pl.BlockSpec((BB, CLASSES), lambda i: (block_off + i, 0))
    out_shape = jax.ShapeDtypeStruct((B, CLASSES), jnp.float32)
    if acc is None:
        body = functools.partial(_mlp_body, None)
        return pl.pallas_call(
            body,
            grid=(nb // BB,),
            in_specs=[pooled_spec] + weight_specs,
            out_specs=out_spec,
            out_shape=out_shape,
        )(pooled, W1, b1, W2, b2)
    return pl.pallas_call(
        _mlp_body,
        grid=(nb // BB,),
        in_specs=[pl.BlockSpec(memory_space=pl.ANY), pooled_spec] + weight_specs,
        out_specs=out_spec,
        out_shape=out_shape,
        input_output_aliases={0: 0},
    )(acc, pooled, W1, b1, W2, b2)


@jax.jit
def _run(idx, emb_table, W1, b1, W2, b2):
    b1r = b1.reshape(1, HIDDEN)
    b2r = b2.reshape(1, CLASSES)
    out = None
    row0 = 0
    for bc in SPLITS:
        pooled = _pool(idx, emb_table, row0 * L, bc)
        out = _mlp_into(out, pooled, W1, b1r, W2, b2r, row0 // BB)
        row0 += bc
    return out


def kernel(x, emb_table, W1, b1, W2, b2):
    idx = x.reshape(-1)
    if idx.dtype != jnp.int32:
        idx = idx.astype(jnp.int32)
    return _run(idx, emb_table, W1, b1, W2, b2)
